# baseline (speedup 1.0000x reference)
import functools

import jax
import jax.numpy as jnp
from jax import lax
from jax.experimental import pallas as pl
from jax.experimental.pallas import tpu as pltpu
from jax.experimental.pallas import tpu_sc as plsc

NUM_ROWS = 1_000_000
EMB = 32
OUT_DIM = 32
BATCH = 16384
SEQ = 50
B_TOTAL = BATCH * SEQ

NC = 2
NS = 16
NW = NC * NS
CHUNK = 1024
SEG = CHUNK // 4
CHUNKS_PER_S = BATCH // CHUNK
N_CHUNKS_TOTAL = SEQ * CHUNKS_PER_S
CHUNKS_PER_W = N_CHUNKS_TOTAL // NW
QH = BATCH // 4


def _make_sc_gather():
    mesh = plsc.VectorSubcoreMesh(core_axis_name="c", subcore_axis_name="s")

    @functools.partial(
        pl.kernel,
        out_type=jax.ShapeDtypeStruct((B_TOTAL, EMB), jnp.float32),
        mesh=mesh,
        scratch_types=[
            pltpu.VMEM((CHUNK,), jnp.int32),
            pltpu.VMEM((CHUNK,), jnp.int32),
            pltpu.VMEM((CHUNK, EMB), jnp.float32),
            pltpu.SemaphoreType.DMA,
        ],
        compiler_params=pltpu.CompilerParams(
            use_tc_tiling_on_sc=False, needs_layout_passes=False),
    )
    def gather_k(idx_hbm, table_hbm, out_hbm, idx_s, idx_v, rows_v, sem):
        wid = lax.axis_index("s") * NC + lax.axis_index("c")

        def chunk_body(i, carry):
            c = wid * CHUNKS_PER_W + i
            s = lax.div(c, CHUNKS_PER_S)
            k = lax.rem(c, CHUNKS_PER_S)
            src_base = s * BATCH + k * SEG
            for j in range(4):
                pltpu.sync_copy(
                    idx_hbm.at[pl.ds(src_base + j * QH, SEG)],
                    idx_s.at[pl.ds(j * SEG, SEG)])

            def reorder(t, carry2):
                lane = lax.iota(jnp.int32, 16)
                const16 = (lane & 3) * SEG + (lane >> 2)
                vals = plsc.load_gather(idx_s, [const16 + t * 4])
                idx_v[pl.ds(t * 16, 16)] = vals
                return carry2

            lax.fori_loop(0, CHUNK // 16, reorder, 0)
            pltpu.async_copy(table_hbm.at[idx_v], rows_v, sem).wait()
            pltpu.sync_copy(rows_v, out_hbm.at[pl.ds(c * CHUNK, CHUNK)])
            return carry

        lax.fori_loop(0, CHUNKS_PER_W, chunk_body, 0)

    return gather_k


_sc_gather = _make_sc_gather()

BCH = BATCH
Q = BCH // 4
ROWS128 = B_TOTAL * EMB // 128


def _linear_relu_body(x_ref, w_ref, b_ref, o_ref):
    x = x_ref[...]
    w = w_ref[...]
    bias = b_ref[...]
    for j in range(4):
        xj = x[:, j * EMB:(j + 1) * EMB]
        yj = lax.dot_general(w, xj, (((1,), (1,)), ((), ())),
                             preferred_element_type=jnp.float32)
        o_ref[0, :, j * Q:(j + 1) * Q] = jnp.maximum(yj + bias, 0.0)


def _tc_linear_relu(x128, w, b2d):
    return pl.pallas_call(
        _linear_relu_body,
        grid=(SEQ,),
        in_specs=[
            pl.BlockSpec((Q, 128), lambda s: (s, 0)),
            pl.BlockSpec((EMB, EMB), lambda s: (0, 0)),
            pl.BlockSpec((OUT_DIM, 1), lambda s: (0, 0)),
        ],
        out_specs=pl.BlockSpec((1, OUT_DIM, BCH), lambda s: (s, 0, 0)),
        out_shape=jax.ShapeDtypeStruct((SEQ, OUT_DIM, BATCH), jnp.float32),
    )(x128, w, b2d)


def kernel(element, table, W, b):
    idx = element.astype(jnp.int32).T.reshape(-1)
    gathered = _sc_gather(idx, table)
    x128 = gathered.reshape(ROWS128, 128)
    yT = _tc_linear_relu(x128, W, b.reshape(OUT_DIM, 1))
    return jnp.transpose(yT, (2, 0, 1))
